# baseline (speedup 1.0000x reference)
import jax
import jax.numpy as jnp
from jax.experimental import pallas as pl
from jax.experimental.pallas import tpu as pltpu

_N, _D, _H, _E, _TOP_K = 32768, 1024, 128, 64, 8
_BLOCK = 2048
_GRID = _N // _BLOCK


def _router_kernel(x_ref, w1_ref, b1_ref, w2_ref, b2_ref, bias_ref,
                   coeffs_ref, mon_ref, cv_ref,
                   sc_scratch, load_acc, mon_acc):
    i = pl.program_id(0)

    @pl.when(i == 0)
    def _init():
        load_acc[...] = jnp.zeros_like(load_acc)
        mon_acc[0, 0] = 0.0
        sc_scratch[1] = jnp.ones((_E, _BLOCK), jnp.float32)

    x = x_ref[...]
    h = x @ w1_ref[...] + b1_ref[...]
    h = h * jax.nn.sigmoid(h)
    logits_t = jax.lax.dot_general(
        w2_ref[...], h, (((0,), (1,)), ((), ()))) + b2_ref[...]
    sc_scratch[jax.lax.rem(i, 2)] = jax.nn.sigmoid(logits_t)

    scores_t = sc_scratch[jax.lax.rem(i + 1, 2)]
    biased = scores_t + bias_ref[...]

    rowf = jax.lax.broadcasted_iota(
        jnp.int32, (_E, _BLOCK), 0).astype(jnp.float32)
    avail = biased
    for _ in range(_TOP_K):
        m = jnp.max(avail, axis=0, keepdims=True)
        key = jnp.where(avail == m, rowf, 128.0)
        idx = jnp.min(key, axis=0, keepdims=True)
        newly = rowf == idx
        avail = jnp.where(newly, -jnp.inf, avail)

    sel = avail == -jnp.inf
    picked = jnp.where(sel, scores_t, 0.0)
    denom = jnp.sum(picked, axis=0, keepdims=True) + 1e-8
    coeffs_t = picked / denom
    coeffs_ref[...] = jnp.transpose(coeffs_t)

    valid = i > 0
    load_acc[...] += jnp.where(
        valid, jnp.sum(jnp.where(sel, 1.0, 0.0), axis=1, keepdims=True), 0.0)
    mon_acc[0, 0] += jnp.where(
        valid, jnp.sum(jnp.max(coeffs_t, axis=0)), 0.0)

    @pl.when(i == _GRID)
    def _fin():
        load = load_acc[...]
        mean = jnp.sum(load) / _E
        var = jnp.sum((load - mean) ** 2) / (_E - 1)
        cv_ref[0, 0] = jnp.sqrt(var) / (mean + 1e-8)
        mon_ref[0, 0] = mon_acc[0, 0] / _N


def kernel(global_features, W1, b1, W2, b2, expert_bias):
    b1r = b1.reshape(1, _H)
    b2r = b2.reshape(_E, 1)
    biasr = expert_bias.reshape(_E, 1)

    coeffs, mon, cv = pl.pallas_call(
        _router_kernel,
        grid=(_GRID + 1,),
        in_specs=[
            pl.BlockSpec((_BLOCK, _D), lambda i: (jnp.minimum(i, _GRID - 1), 0)),
            pl.BlockSpec((_D, _H), lambda i: (0, 0)),
            pl.BlockSpec((1, _H), lambda i: (0, 0)),
            pl.BlockSpec((_H, _E), lambda i: (0, 0)),
            pl.BlockSpec((_E, 1), lambda i: (0, 0)),
            pl.BlockSpec((_E, 1), lambda i: (0, 0)),
        ],
        out_specs=[
            pl.BlockSpec((_BLOCK, _E), lambda i: (jnp.maximum(i, 1) - 1, 0)),
            pl.BlockSpec(memory_space=pltpu.SMEM),
            pl.BlockSpec(memory_space=pltpu.SMEM),
        ],
        out_shape=[
            jax.ShapeDtypeStruct((_N, _E), jnp.float32),
            jax.ShapeDtypeStruct((1, 1), jnp.float32),
            jax.ShapeDtypeStruct((1, 1), jnp.float32),
        ],
        scratch_shapes=[
            pltpu.VMEM((2, _E, _BLOCK), jnp.float32),
            pltpu.VMEM((_E, 1), jnp.float32),
            pltpu.SMEM((1, 1), jnp.float32),
        ],
        compiler_params=pltpu.CompilerParams(
            dimension_semantics=("arbitrary",),
        ),
    )(global_features, W1, b1r, W2, b2r, biasr)

    return (coeffs, mon[0, 0], cv[0, 0])


# speedup vs baseline: 1.0958x; 1.0958x over previous
import jax
import jax.numpy as jnp
from jax.experimental import pallas as pl
from jax.experimental.pallas import tpu as pltpu

_N, _D, _H, _E, _TOP_K = 32768, 1024, 128, 64, 8
_BLOCK = 4096
_GRID = _N // _BLOCK


def _router_kernel(x_ref, w1_ref, b1_ref, w2_ref, b2_ref, bias_ref, rowf_ref,
                   coeffs_ref, mon_ref, cv_ref, load_acc, mon_acc):
    i = pl.program_id(0)

    @pl.when(i == 0)
    def _init():
        load_acc[...] = jnp.zeros_like(load_acc)
        mon_acc[0, 0] = 0.0

    x = x_ref[...]
    h = x @ w1_ref[...] + b1_ref[...]
    h = h * jax.nn.sigmoid(h)
    logits = h @ w2_ref[...] + b2_ref[...]
    scores_t = jnp.transpose(jax.nn.sigmoid(logits))
    biased = scores_t + bias_ref[...]

    rowf = rowf_ref[...]
    avail = biased
    for _ in range(_TOP_K):
        m = jnp.max(avail, axis=0, keepdims=True)
        key = jnp.where(avail == m, rowf, 128.0)
        idx = jnp.min(key, axis=0, keepdims=True)
        newly = rowf == idx
        avail = jnp.where(newly, -jnp.inf, avail)

    sel = avail == -jnp.inf
    picked = jnp.where(sel, scores_t, 0.0)
    denom = jnp.sum(picked, axis=0, keepdims=True) + 1e-8
    coeffs_t = picked / denom
    coeffs_ref[...] = jnp.transpose(coeffs_t)

    load_acc[...] += jnp.sum(jnp.where(sel, 1.0, 0.0), axis=1, keepdims=True)
    mon_acc[0, 0] += jnp.sum(jnp.max(coeffs_t, axis=0))

    @pl.when(i == _GRID - 1)
    def _fin():
        load = load_acc[...]
        mean = jnp.sum(load) / _E
        var = jnp.sum((load - mean) ** 2) / (_E - 1)
        cv_ref[0, 0] = jnp.sqrt(var) / (mean + 1e-8)
        mon_ref[0, 0] = mon_acc[0, 0] / _N


def kernel(global_features, W1, b1, W2, b2, expert_bias):
    b1r = b1.reshape(1, _H)
    b2r = b2.reshape(1, _E)
    biasr = expert_bias.reshape(_E, 1)
    rowm = jnp.broadcast_to(jnp.arange(_E, dtype=jnp.float32)[:, None], (_E, _BLOCK))

    coeffs, mon, cv = pl.pallas_call(
        _router_kernel,
        grid=(_GRID,),
        in_specs=[
            pl.BlockSpec((_BLOCK, _D), lambda i: (i, 0)),
            pl.BlockSpec((_D, _H), lambda i: (0, 0)),
            pl.BlockSpec((1, _H), lambda i: (0, 0)),
            pl.BlockSpec((_H, _E), lambda i: (0, 0)),
            pl.BlockSpec((1, _E), lambda i: (0, 0)),
            pl.BlockSpec((_E, 1), lambda i: (0, 0)),
            pl.BlockSpec((_E, _BLOCK), lambda i: (0, 0)),
        ],
        out_specs=[
            pl.BlockSpec((_BLOCK, _E), lambda i: (i, 0)),
            pl.BlockSpec(memory_space=pltpu.SMEM),
            pl.BlockSpec(memory_space=pltpu.SMEM),
        ],
        out_shape=[
            jax.ShapeDtypeStruct((_N, _E), jnp.float32),
            jax.ShapeDtypeStruct((1, 1), jnp.float32),
            jax.ShapeDtypeStruct((1, 1), jnp.float32),
        ],
        scratch_shapes=[
            pltpu.VMEM((_E, 1), jnp.float32),
            pltpu.SMEM((1, 1), jnp.float32),
        ],
        compiler_params=pltpu.CompilerParams(
            dimension_semantics=("arbitrary",),
        ),
    )(global_features, W1, b1r, W2, b2r, biasr, rowm)

    return (coeffs, mon[0, 0], cv[0, 0])
